# baseline (speedup 1.0000x reference)
import functools

import jax
import jax.numpy as jnp
from jax import lax
from jax.experimental import pallas as pl
from jax.experimental.pallas import tpu as pltpu
from jax.experimental.pallas import tpu_sc as plsc


@functools.partial(jax.jit, static_argnames=("num_rows", "d", "ch", "nch"))
def _sc_gather(idx, table, *, num_rows, d, ch, nch):
    nw = 32
    bpw = num_rows // nw

    mesh = plsc.VectorSubcoreMesh(core_axis_name="c", subcore_axis_name="s")

    @functools.partial(
        pl.kernel,
        mesh=mesh,
        out_type=jax.ShapeDtypeStruct((num_rows, d), jnp.float32),
        scratch_types=[
            pltpu.VMEM((ch,), jnp.int32),
            pltpu.VMEM((ch, d), jnp.float32),
            pltpu.SemaphoreType.DMA,
        ],
        compiler_params=pltpu.CompilerParams(use_tc_tiling_on_sc=False),
    )
    def body(tokens_hbm, table_hbm, out_hbm, idx_v, rows_v, sem):
        wid = lax.axis_index("s") * 2 + lax.axis_index("c")
        row0 = wid * bpw

        def chunk(g, carry):
            off = row0 + g * ch
            pltpu.sync_copy(tokens_hbm.at[pl.ds(off, ch)], idx_v)
            pltpu.async_copy(table_hbm.at[idx_v], rows_v, sem).wait()
            pltpu.sync_copy(rows_v, out_hbm.at[pl.ds(off, ch)])
            return carry

        lax.fori_loop(0, nch, chunk, 0)

    return body(idx, table)


def kernel(tokens, table):
    b, t = tokens.shape
    num_rows = b * t
    d = table.shape[1]
    idx = tokens.astype(jnp.int32).reshape(num_rows)
    ch = 512
    nch = num_rows // 32 // ch
    out = _sc_gather(idx, table, num_rows=num_rows, d=d, ch=ch, nch=nch)
    return out.reshape(b, t, d)


# speedup vs baseline: 1.0434x; 1.0434x over previous
import functools

import jax
import jax.numpy as jnp
from jax import lax
from jax.experimental import pallas as pl
from jax.experimental.pallas import tpu as pltpu
from jax.experimental.pallas import tpu_sc as plsc

_NBUF = 4
_LOOKAHEAD = 2


@functools.partial(jax.jit, static_argnames=("num_rows", "d", "ch", "nch"))
def _sc_gather(idx, table, *, num_rows, d, ch, nch):
    nw = 32
    bpw = num_rows // nw

    mesh = plsc.VectorSubcoreMesh(core_axis_name="c", subcore_axis_name="s")

    @functools.partial(
        pl.kernel,
        mesh=mesh,
        out_type=jax.ShapeDtypeStruct((num_rows, d), jnp.float32),
        scratch_types=[
            pltpu.VMEM((bpw,), jnp.int32),
            [pltpu.VMEM((ch, d), jnp.float32) for _ in range(_NBUF)],
            [pltpu.SemaphoreType.DMA for _ in range(_NBUF)],
            [pltpu.SemaphoreType.DMA for _ in range(_NBUF)],
        ],
        compiler_params=pltpu.CompilerParams(use_tc_tiling_on_sc=False),
    )
    def body(tokens_hbm, table_hbm, out_hbm, idx_v, bufs, gsems, wsems):
        wid = lax.axis_index("s") * 2 + lax.axis_index("c")
        row0 = wid * bpw

        def fire_gather_dyn(g, b):
            pltpu.async_copy(
                table_hbm.at[idx_v.at[pl.ds(g * ch, ch)]], bufs[b], gsems[b]
            )

        def drain_gather(b):
            pltpu.make_async_copy(
                out_hbm.at[pl.ds(0, ch)], bufs[b], gsems[b]
            ).wait()

        def fire_write(g, b):
            pltpu.async_copy(
                bufs[b], out_hbm.at[pl.ds(row0 + g * ch, ch)], wsems[b]
            )

        def drain_write(b):
            pltpu.make_async_copy(
                bufs[b], out_hbm.at[pl.ds(0, ch)], wsems[b]
            ).wait()

        pltpu.sync_copy(tokens_hbm.at[pl.ds(row0, bpw)], idx_v)

        fire_gather_dyn(0, 0)
        fire_gather_dyn(1, 1)
        drain_gather(0)
        fire_write(0, 0)
        fire_gather_dyn(2, 2)
        drain_gather(1)
        fire_write(1, 1)
        fire_gather_dyn(3, 3)

        def step(g, carry):
            for i in range(_NBUF):
                b = i
                gg = g + i
                drain_write(b)
                fire_gather_dyn(gg, b)
                bw = (i + _NBUF - _LOOKAHEAD) % _NBUF
                drain_gather(bw)
                fire_write(gg - _LOOKAHEAD, bw)
            return carry

        lax.fori_loop(0, (nch - _NBUF) // _NBUF, lambda p, c: step(_NBUF + p * _NBUF, c), 0)

        drain_gather((nch - 2) % _NBUF)
        fire_write(nch - 2, (nch - 2) % _NBUF)
        drain_gather((nch - 1) % _NBUF)
        fire_write(nch - 1, (nch - 1) % _NBUF)
        for b in range(_NBUF):
            drain_write(b)

    return body(idx, table)


def kernel(tokens, table):
    b, t = tokens.shape
    num_rows = b * t
    d = table.shape[1]
    idx = tokens.astype(jnp.int32).reshape(num_rows)
    ch = 400
    nch = num_rows // 32 // ch
    out = _sc_gather(idx, table, num_rows=num_rows, d=d, ch=ch, nch=nch)
    return out.reshape(b, t, d)
